# Initial kernel scaffold; baseline (speedup 1.0000x reference)
import jax
import jax.numpy as jnp
from jax.experimental import pallas as pl


def kernel(indices, table):
    raise NotImplementedError("write your pallas kernel here")


# speedup vs baseline: 1.5272x; 1.5272x over previous
import functools

import jax
import jax.numpy as jnp
from jax import lax
from jax.experimental import pallas as pl
from jax.experimental.pallas import tpu as pltpu
from jax.experimental.pallas import tpu_sc as plsc

_VOCAB = 100000
_DIM = 1024
_BATCH = 16384
_NC = 2
_NS = 16
_NW = _NC * _NS
_BPW = _BATCH // _NW
_CH = 64
_NCHUNK = _BPW // _CH

_mesh = plsc.VectorSubcoreMesh(core_axis_name="c", subcore_axis_name="s")


@functools.partial(
    pl.kernel,
    mesh=_mesh,
    out_type=jax.ShapeDtypeStruct((_BATCH, _DIM), jnp.float32),
    scratch_types=[
        pltpu.VMEM((_BPW,), jnp.int32),
        pltpu.VMEM((_CH, _DIM), jnp.float32),
        pltpu.SemaphoreType.DMA,
    ],
)
def _sc_gather(table_hbm, idx_hbm, out_hbm, idx_v, rows_v, sem):
    wid = lax.axis_index("s") * _NC + lax.axis_index("c")
    base = wid * _BPW
    pltpu.sync_copy(idx_hbm.at[pl.ds(base, _BPW)], idx_v)
    for ci in range(_NCHUNK):
        cb = ci * _CH
        pltpu.async_copy(
            table_hbm.at[idx_v.at[pl.ds(cb, _CH)]], rows_v, sem
        ).wait()
        pltpu.sync_copy(rows_v, out_hbm.at[pl.ds(base + cb, _CH)])


def kernel(indices, table):
    return _sc_gather(table, indices.astype(jnp.int32))


# speedup vs baseline: 1.6412x; 1.0746x over previous
import functools

import jax
import jax.numpy as jnp
from jax import lax
from jax.experimental import pallas as pl
from jax.experimental.pallas import tpu as pltpu
from jax.experimental.pallas import tpu_sc as plsc

_VOCAB = 100000
_DIM = 1024
_BATCH = 16384
_NC = 2
_NS = 16
_NW = _NC * _NS
_BPW = _BATCH // _NW
_CH = 32
_NCHUNK = _BPW // _CH
_NBUF = 3

_mesh = plsc.VectorSubcoreMesh(core_axis_name="c", subcore_axis_name="s")


@functools.partial(
    pl.kernel,
    mesh=_mesh,
    out_type=jax.ShapeDtypeStruct((_BATCH, _DIM), jnp.float32),
    scratch_types=[
        pltpu.VMEM((_BPW,), jnp.int32),
        pltpu.VMEM((_NBUF, _CH, _DIM), jnp.float32),
        pltpu.SemaphoreType.DMA,
        pltpu.SemaphoreType.DMA,
    ],
)
def _sc_gather(table_hbm, idx_hbm, out_hbm, idx_v, rows_v, gsem, wsem):
    wid = lax.axis_index("s") * _NC + lax.axis_index("c")
    base = wid * _BPW
    pltpu.sync_copy(idx_hbm.at[pl.ds(base, _BPW)], idx_v)

    def start_gather(ci):
        return pltpu.async_copy(
            table_hbm.at[idx_v.at[pl.ds(ci * _CH, _CH)]],
            rows_v.at[ci % _NBUF],
            gsem,
        )

    gd = [None] * _NCHUNK
    wd = [None] * _NCHUNK
    for ci in range(min(_NBUF, _NCHUNK)):
        gd[ci] = start_gather(ci)
    for ci in range(_NCHUNK):
        gd[ci].wait()
        wd[ci] = pltpu.async_copy(
            rows_v.at[ci % _NBUF], out_hbm.at[pl.ds(base + ci * _CH, _CH)], wsem
        )
        nxt = ci + _NBUF
        if nxt < _NCHUNK:
            wd[ci].wait()
            gd[nxt] = start_gather(nxt)
    for ci in range(max(0, _NCHUNK - _NBUF), _NCHUNK):
        wd[ci].wait()


def kernel(indices, table):
    return _sc_gather(table, indices.astype(jnp.int32))
